# baseline (speedup 1.0000x reference)
import functools

import jax
import jax.numpy as jnp
from jax import lax
from jax.experimental import pallas as pl
from jax.experimental.pallas import tpu as pltpu
from jax.experimental.pallas import tpu_sc as plsc

B = 32; T = 20; G = 300; H = 512
R = 36; IMG = 2048; REL = 7
SN = 40; SD = 300; SE = 60; SRD = 300
FN = 100; FD = 1024
PN = 1024; PR = 512; PS = 512; PF = 512

NC, NS = 2, 16
NW = NC * NS
QTOT = B * T
PER_W = 24
QPAD = NW * PER_W


GP = 384


def _sc_gather_rows(glove_pad, idx):
    mesh = plsc.VectorSubcoreMesh(core_axis_name="c", subcore_axis_name="s")

    @functools.partial(
        pl.kernel, mesh=mesh,
        out_type=jax.ShapeDtypeStruct((QPAD, GP), jnp.float32),
        scratch_types=[
            pltpu.VMEM((PER_W,), jnp.int32),
            pltpu.VMEM((PER_W, GP), jnp.float32),
            pltpu.SemaphoreType.DMA,
        ],
    )
    def gather_k(glove_hbm, idx_hbm, out_hbm, idx_v, rows_v, sem):
        wid = lax.axis_index("s") * NC + lax.axis_index("c")
        base = wid * PER_W
        pltpu.sync_copy(idx_hbm.at[pl.ds(base, PER_W)], idx_v)
        pltpu.async_copy(glove_hbm.at[idx_v], rows_v, sem).wait()
        pltpu.sync_copy(rows_v, out_hbm.at[pl.ds(base, PER_W)])

    return gather_k(glove_pad, idx)


def _lstm_body(qe_ref, wih_ref, whh_ref, bias_ref, lens_ref, q_ref, xg_ref):
    xg_ref[...] = (
        jnp.dot(qe_ref[pl.ds(0, QTOT), :], wih_ref[...],
                preferred_element_type=jnp.float32)
        + bias_ref[...]
    )

    def step(t, carry):
        h, c, hlast = carry
        g = xg_ref[pl.ds(t * B, B), :] + jnp.dot(
            h, whh_ref[...], preferred_element_type=jnp.float32)
        i = jax.nn.sigmoid(g[:, :H])
        f = jax.nn.sigmoid(g[:, H:2 * H])
        gg = jnp.tanh(g[:, 2 * H:3 * H])
        o = jax.nn.sigmoid(g[:, 3 * H:])
        c = f * c + i * gg
        h = o * jnp.tanh(c)
        msk = (lens_ref[:, :1] - 1) == t
        hlast = jnp.where(msk, h, hlast)
        return (h, c, hlast)

    z = jnp.zeros((B, H), jnp.float32)
    _, _, hlast = lax.fori_loop(0, T, step, (z, z, z))
    q_ref[...] = hlast


def _lstm_call(qe, wihT, whhT, bias, lens):
    return pl.pallas_call(
        _lstm_body,
        out_shape=jax.ShapeDtypeStruct((B, H), jnp.float32),
        scratch_shapes=[pltpu.VMEM((QTOT, 4 * H), jnp.float32)],
    )(qe, wihT, whhT, bias, lens)


def _mm_body(x_ref, w_ref, b_ref, o_ref):
    o_ref[...] = (
        jnp.dot(x_ref[...], w_ref[...], preferred_element_type=jnp.float32)
        + b_ref[...]
    )


def _proj(x, w, b, bm):
    m, k = x.shape
    n = w.shape[1]
    return pl.pallas_call(
        _mm_body,
        grid=(m // bm,),
        in_specs=[
            pl.BlockSpec((bm, k), lambda i: (i, 0)),
            pl.BlockSpec((k, n), lambda i: (0, 0)),
            pl.BlockSpec((1, n), lambda i: (0, 0)),
        ],
        out_specs=pl.BlockSpec((bm, n), lambda i: (i, 0)),
        out_shape=jax.ShapeDtypeStruct((m, n), jnp.float32),
    )(x, w, b)


def _att_body(q_ref, pi_ref, rel_ref, ss_ref, ee_ref, fn_ref,
              wq_vn, wq_vr, wrT, br_vr, wq_sn, wq_sr, wq_fn,
              bq_vn, bq_vr, bq_sn, bq_sr, bq_fn,
              w_vn, w_vr, w_sn, w_sr, w_fn,
              o_vn, o_vr, o_sn, o_sr, o_fn):
    q = q_ref[0]

    def head(proj, wq, bq, wv):
        pq = jnp.dot(q, wq, preferred_element_type=jnp.float32) + bq
        s = jnp.tanh(pq + proj)
        return jnp.sum(s * wv, axis=-1, keepdims=True)

    def smax(x):
        m = jnp.max(x, axis=0, keepdims=True)
        e = jnp.exp(x - m)
        return e / jnp.sum(e, axis=0, keepdims=True)

    o_vn[0] = smax(head(pi_ref[0], wq_vn[...], bq_vn[...], w_vn[...]))
    rr = jnp.dot(rel_ref[0], wrT[...],
                 preferred_element_type=jnp.float32) + br_vr[...]
    o_vr[0] = head(rr, wq_vr[...], bq_vr[...], w_vr[...])
    o_sn[0] = smax(head(ss_ref[0], wq_sn[...], bq_sn[...], w_sn[...]))
    o_sr[0] = smax(head(ee_ref[0], wq_sr[...], bq_sr[...], w_sr[...]))
    o_fn[0] = smax(head(fn_ref[0], wq_fn[...], bq_fn[...], w_fn[...]))


def _att_call(q3, pi, rel8, ss, ee, fn, ws):
    def bspec(shape):
        return pl.BlockSpec((1,) + shape[1:], lambda b: (b,) + (0,) * (len(shape) - 1))

    def cspec(shape):
        return pl.BlockSpec(shape, lambda b: (0,) * len(shape))

    ins = [q3, pi, rel8, ss, ee, fn] + ws
    in_specs = [bspec(x.shape) for x in (q3, pi, rel8, ss, ee, fn)]
    in_specs += [cspec(w.shape) for w in ws]
    out_shapes = [
        jax.ShapeDtypeStruct((B, R, 1), jnp.float32),
        jax.ShapeDtypeStruct((B, R * R, 1), jnp.float32),
        jax.ShapeDtypeStruct((B, SN, 1), jnp.float32),
        jax.ShapeDtypeStruct((B, SE, 1), jnp.float32),
        jax.ShapeDtypeStruct((B, FN, 1), jnp.float32),
    ]
    out_specs = [bspec(s.shape) for s in out_shapes]
    return pl.pallas_call(
        _att_body,
        grid=(B,),
        in_specs=in_specs,
        out_specs=out_specs,
        out_shape=out_shapes,
    )(*ins)


def kernel(questions, question_length, images, img_relations,
           sem_node_features, sem_edge_features, fact_node_features,
           glove, params):
    p = params
    f32 = jnp.float32

    idx = questions.astype(jnp.int32).T.reshape(-1)
    idx = jnp.concatenate([idx, jnp.zeros((QPAD - QTOT,), jnp.int32)])
    glove_pad = jnp.pad(glove, ((0, 0), (0, GP - G)))
    qe = _sc_gather_rows(glove_pad, idx)

    bias = (p["bih"] + p["bhh"]).reshape(1, 4 * H)
    lens = jnp.broadcast_to(
        question_length.astype(jnp.int32)[:, None], (B, 128))
    wihT = jnp.pad(p["Wih"].T, ((0, GP - G), (0, 0)))
    q = _lstm_call(qe, wihT, p["Whh"].T, bias, lens)

    pi = _proj(images.reshape(B * R, IMG), p["vn_Wi"].T,
               p["vn_bi"].reshape(1, PN), 128).reshape(B, R, PN)
    fn = _proj(fact_node_features.reshape(B * FN, FD), p["fn_Wn"].T,
               p["fn_bn"].reshape(1, PF), 128).reshape(B, FN, PF)
    ss = _proj(sem_node_features.reshape(B * SN, SD), p["sn_Ws"].T,
               p["sn_bs"].reshape(1, PS), 128).reshape(B, SN, PS)
    ee = _proj(sem_edge_features.reshape(B * SE, SRD), p["sr_Wr"].T,
               p["sr_br"].reshape(1, PR), 128).reshape(B, SE, PR)

    rel8 = jnp.concatenate(
        [img_relations.reshape(B, R * R, REL),
         jnp.zeros((B, R * R, 1), f32)], axis=-1)
    wrT8 = jnp.concatenate(
        [p["vr_Wr"].T, jnp.zeros((1, PR), f32)], axis=0)

    ws = [
        p["vn_Wq"].T, p["vr_Wq"].T, wrT8, p["vr_br"].reshape(1, PR),
        p["sn_Wq"].T, p["sr_Wq"].T, p["fn_Wq"].T,
        p["vn_bq"].reshape(1, PN), p["vr_bq"].reshape(1, PR),
        p["sn_bq"].reshape(1, PS), p["sr_bq"].reshape(1, PR),
        p["fn_bq"].reshape(1, PF),
        p["vn_w"].reshape(1, PN), p["vr_w"].reshape(1, PR),
        p["sn_w"].reshape(1, PS), p["sr_w"].reshape(1, PR),
        p["fn_w"].reshape(1, PF),
    ]
    o_vn, o_vr, o_sn, o_sr, o_fn = _att_call(
        q.reshape(B, 1, H), pi, rel8, ss, ee, fn, ws)

    vis_node_att = o_vn.reshape(B, R)
    vis_rel_att = o_vr.reshape(B, R, R) + p["vr_b"][0]
    sem_node_att = o_sn.reshape(B, SN)
    sem_rel_att = o_sr.reshape(B, SE)
    fact_node_att = o_fn.reshape(B, FN)
    return vis_node_att, vis_rel_att, sem_node_att, sem_rel_att, fact_node_att


# speedup vs baseline: 1.1718x; 1.1718x over previous
import functools

import jax
import jax.numpy as jnp
from jax import lax
from jax.experimental import pallas as pl
from jax.experimental.pallas import tpu as pltpu
from jax.experimental.pallas import tpu_sc as plsc

B = 32; T = 20; G = 300; H = 512
R = 36; IMG = 2048; REL = 7
SN = 40; SD = 300; SE = 60; SRD = 300
FN = 100; FD = 1024
PN = 1024; PR = 512; PS = 512; PF = 512

NC, NS = 2, 16
NW = NC * NS
QTOT = B * T
PER_W = 24
QPAD = NW * PER_W
GP = 384
V = 20000


def _dot_t(x, w):
    return lax.dot_general(x, w, (((1,), (1,)), ((), ())),
                           preferred_element_type=jnp.float32)


def _pad_body(x_ref, o_ref):
    o_ref[...] = jnp.concatenate(
        [x_ref[...], jnp.zeros((x_ref.shape[0], GP - G), jnp.float32)], axis=1)


def _pad_glove(glove):
    bm = 400
    return pl.pallas_call(
        _pad_body,
        grid=(V // bm,),
        in_specs=[pl.BlockSpec((bm, G), lambda i: (i, 0))],
        out_specs=pl.BlockSpec((bm, GP), lambda i: (i, 0)),
        out_shape=jax.ShapeDtypeStruct((V, GP), jnp.float32),
    )(glove)


def _sc_gather_rows(glove_pad, idx):
    mesh = plsc.VectorSubcoreMesh(core_axis_name="c", subcore_axis_name="s")

    @functools.partial(
        pl.kernel, mesh=mesh,
        out_type=jax.ShapeDtypeStruct((QPAD, GP), jnp.float32),
        scratch_types=[
            pltpu.VMEM((PER_W,), jnp.int32),
            pltpu.VMEM((PER_W, GP), jnp.float32),
            pltpu.SemaphoreType.DMA,
        ],
    )
    def gather_k(glove_hbm, idx_hbm, out_hbm, idx_v, rows_v, sem):
        wid = lax.axis_index("s") * NC + lax.axis_index("c")
        base = wid * PER_W
        pltpu.sync_copy(idx_hbm.at[pl.ds(base, PER_W)], idx_v)
        pltpu.async_copy(glove_hbm.at[idx_v], rows_v, sem).wait()
        pltpu.sync_copy(rows_v, out_hbm.at[pl.ds(base, PER_W)])

    return gather_k(glove_pad, idx)


def _lstm_body(qe_ref, wih_ref, whh_ref, bias_ref, lens_ref, q_ref, xg_ref):
    xg_ref[...] = (
        _dot_t(qe_ref[pl.ds(0, QTOT), pl.ds(0, G)], wih_ref[...])
        + bias_ref[...]
    )

    def step(t, carry):
        h, c, hlast = carry
        g = xg_ref[pl.ds(t * B, B), :] + _dot_t(h, whh_ref[...])
        i = jax.nn.sigmoid(g[:, :H])
        f = jax.nn.sigmoid(g[:, H:2 * H])
        gg = jnp.tanh(g[:, 2 * H:3 * H])
        o = jax.nn.sigmoid(g[:, 3 * H:])
        c = f * c + i * gg
        h = o * jnp.tanh(c)
        msk = (lens_ref[:, :1] - 1) == t
        hlast = jnp.where(msk, h, hlast)
        return (h, c, hlast)

    z = jnp.zeros((B, H), jnp.float32)
    _, _, hlast = lax.fori_loop(0, T, step, (z, z, z))
    q_ref[...] = hlast


def _lstm_call(qe, wih, whh, bias, lens):
    return pl.pallas_call(
        _lstm_body,
        out_shape=jax.ShapeDtypeStruct((B, H), jnp.float32),
        scratch_shapes=[pltpu.VMEM((QTOT, 4 * H), jnp.float32)],
    )(qe, wih, whh, bias, lens)


def _mm_body(x_ref, w_ref, b_ref, o_ref):
    o_ref[...] = _dot_t(x_ref[...], w_ref[...]) + b_ref[...]


def _proj(x, w, b, bm):
    m, k = x.shape
    n = w.shape[0]
    return pl.pallas_call(
        _mm_body,
        grid=(m // bm,),
        in_specs=[
            pl.BlockSpec((bm, k), lambda i: (i, 0)),
            pl.BlockSpec((n, k), lambda i: (0, 0)),
            pl.BlockSpec((1, n), lambda i: (0, 0)),
        ],
        out_specs=pl.BlockSpec((bm, n), lambda i: (i, 0)),
        out_shape=jax.ShapeDtypeStruct((m, n), jnp.float32),
    )(x, w, b)


def _att_body(q_ref, pi_ref, rel_ref, ss_ref, ee_ref, fn_ref,
              wq_vn, wq_vr, wr, br_vr, wq_sn, wq_sr, wq_fn,
              bq_vn, bq_vr, bq_sn, bq_sr, bq_fn,
              w_vn, w_vr, w_sn, w_sr, w_fn,
              o_vn, o_vr, o_sn, o_sr, o_fn):
    q = q_ref[0]

    def head(proj, wq, bq, wv):
        pq = _dot_t(q, wq) + bq
        s = jnp.tanh(pq + proj)
        return jnp.sum(s * wv, axis=-1, keepdims=True)

    def smax(x):
        m = jnp.max(x, axis=0, keepdims=True)
        e = jnp.exp(x - m)
        return e / jnp.sum(e, axis=0, keepdims=True)

    o_vn[0] = smax(head(pi_ref[0], wq_vn[...], bq_vn[...], w_vn[...]))
    rr = _dot_t(rel_ref[0], wr[...]) + br_vr[...]
    o_vr[0] = head(rr, wq_vr[...], bq_vr[...], w_vr[...])
    o_sn[0] = smax(head(ss_ref[0], wq_sn[...], bq_sn[...], w_sn[...]))
    o_sr[0] = smax(head(ee_ref[0], wq_sr[...], bq_sr[...], w_sr[...]))
    o_fn[0] = smax(head(fn_ref[0], wq_fn[...], bq_fn[...], w_fn[...]))


def _att_call(q3, pi, rel, ss, ee, fn, ws):
    def bspec(shape):
        return pl.BlockSpec((1,) + shape[1:], lambda b: (b,) + (0,) * (len(shape) - 1))

    def cspec(shape):
        return pl.BlockSpec(shape, lambda b: (0,) * len(shape))

    ins = [q3, pi, rel, ss, ee, fn] + ws
    in_specs = [bspec(x.shape) for x in (q3, pi, rel, ss, ee, fn)]
    in_specs += [cspec(w.shape) for w in ws]
    out_shapes = [
        jax.ShapeDtypeStruct((B, R, 1), jnp.float32),
        jax.ShapeDtypeStruct((B, R * R, 1), jnp.float32),
        jax.ShapeDtypeStruct((B, SN, 1), jnp.float32),
        jax.ShapeDtypeStruct((B, SE, 1), jnp.float32),
        jax.ShapeDtypeStruct((B, FN, 1), jnp.float32),
    ]
    out_specs = [bspec(s.shape) for s in out_shapes]
    return pl.pallas_call(
        _att_body,
        grid=(B,),
        in_specs=in_specs,
        out_specs=out_specs,
        out_shape=out_shapes,
    )(*ins)


def kernel(questions, question_length, images, img_relations,
           sem_node_features, sem_edge_features, fact_node_features,
           glove, params):
    p = params

    idx = questions.astype(jnp.int32).T.reshape(-1)
    idx = jnp.concatenate([idx, jnp.zeros((QPAD - QTOT,), jnp.int32)])
    qe = _sc_gather_rows(_pad_glove(glove), idx)

    bias = (p["bih"] + p["bhh"]).reshape(1, 4 * H)
    lens = jnp.broadcast_to(
        question_length.astype(jnp.int32)[:, None], (B, 128))
    q = _lstm_call(qe, p["Wih"], p["Whh"], bias, lens)

    pi = _proj(images.reshape(B * R, IMG), p["vn_Wi"],
               p["vn_bi"].reshape(1, PN), 128).reshape(B, R, PN)
    fn = _proj(fact_node_features.reshape(B * FN, FD), p["fn_Wn"],
               p["fn_bn"].reshape(1, PF), 128).reshape(B, FN, PF)
    ss = _proj(sem_node_features.reshape(B * SN, SD), p["sn_Ws"],
               p["sn_bs"].reshape(1, PS), 128).reshape(B, SN, PS)
    ee = _proj(sem_edge_features.reshape(B * SE, SRD), p["sr_Wr"],
               p["sr_br"].reshape(1, PR), 128).reshape(B, SE, PR)

    rel = img_relations.reshape(B, R * R, REL)

    ws = [
        p["vn_Wq"], p["vr_Wq"], p["vr_Wr"], p["vr_br"].reshape(1, PR),
        p["sn_Wq"], p["sr_Wq"], p["fn_Wq"],
        p["vn_bq"].reshape(1, PN), p["vr_bq"].reshape(1, PR),
        p["sn_bq"].reshape(1, PS), p["sr_bq"].reshape(1, PR),
        p["fn_bq"].reshape(1, PF),
        p["vn_w"].reshape(1, PN), p["vr_w"].reshape(1, PR),
        p["sn_w"].reshape(1, PS), p["sr_w"].reshape(1, PR),
        p["fn_w"].reshape(1, PF),
    ]
    o_vn, o_vr, o_sn, o_sr, o_fn = _att_call(
        q.reshape(B, 1, H), pi, rel, ss, ee, fn, ws)

    vis_node_att = o_vn.reshape(B, R)
    vis_rel_att = o_vr.reshape(B, R, R) + p["vr_b"][0]
    sem_node_att = o_sn.reshape(B, SN)
    sem_rel_att = o_sr.reshape(B, SE)
    fact_node_att = o_fn.reshape(B, FN)
    return vis_node_att, vis_rel_att, sem_node_att, sem_rel_att, fact_node_att


# speedup vs baseline: 2.0969x; 1.7894x over previous
import functools

import jax
import jax.numpy as jnp
from jax import lax
from jax.experimental import pallas as pl
from jax.experimental.pallas import tpu as pltpu
from jax.experimental.pallas import tpu_sc as plsc

B = 32; T = 20; G = 300; H = 512
R = 36; IMG = 2048; REL = 7
SN = 40; SD = 300; SE = 60; SRD = 300
FN = 100; FD = 1024
PN = 1024; PR = 512; PS = 512; PF = 512

NC, NS = 2, 16
NW = NC * NS
QTOT = B * T
PER_W = 24
QPAD = NW * PER_W
GP = 384
V = 20000
BB = 8


def _dot_t(x, w):
    return lax.dot_general(x, w, (((1,), (1,)), ((), ())),
                           preferred_element_type=jnp.float32)


def _bf(x):
    return x.astype(jnp.bfloat16)


def _pad_body(xt_ref, o_ref):
    blk = jnp.swapaxes(xt_ref[...], 0, 1)
    o_ref[...] = jnp.concatenate(
        [blk, jnp.zeros((blk.shape[0], GP - G), jnp.float32)], axis=1)


def _pad_glove_t(glove_t):
    bm = 512
    grid = (V + bm - 1) // bm
    return pl.pallas_call(
        _pad_body,
        grid=(grid,),
        in_specs=[pl.BlockSpec((G, bm), lambda i: (0, i))],
        out_specs=pl.BlockSpec((bm, GP), lambda i: (i, 0)),
        out_shape=jax.ShapeDtypeStruct((V, GP), jnp.float32),
    )(glove_t)


def _sc_gather_rows(glove_pad, idx):
    mesh = plsc.VectorSubcoreMesh(core_axis_name="c", subcore_axis_name="s")

    @functools.partial(
        pl.kernel, mesh=mesh,
        out_type=jax.ShapeDtypeStruct((QPAD, GP), jnp.float32),
        scratch_types=[
            pltpu.VMEM((PER_W,), jnp.int32),
            pltpu.VMEM((PER_W, GP), jnp.float32),
            pltpu.SemaphoreType.DMA,
        ],
    )
    def gather_k(glove_hbm, idx_hbm, out_hbm, idx_v, rows_v, sem):
        wid = lax.axis_index("s") * NC + lax.axis_index("c")
        base = wid * PER_W
        pltpu.sync_copy(idx_hbm.at[pl.ds(base, PER_W)], idx_v)
        pltpu.async_copy(glove_hbm.at[idx_v], rows_v, sem).wait()
        pltpu.sync_copy(rows_v, out_hbm.at[pl.ds(base, PER_W)])

    return gather_k(glove_pad, idx)


def _lstm_body(qe_ref, wihT_ref, whh_ref, bias_ref, lens_ref,
               wq_vn, bq_vn, wq_vr, bq_vr, wq_sn, bq_sn,
               wq_sr, bq_sr, wq_fn, bq_fn,
               pq_ref, rq_ref, sq_ref, eq_ref, fq_ref, xg_ref):
    xg_ref[...] = (
        jnp.dot(qe_ref[pl.ds(0, QTOT), pl.ds(0, G)], wihT_ref[...],
                preferred_element_type=jnp.float32)
        + bias_ref[...]
    )

    def step(t, carry):
        h, c, hlast = carry
        g = xg_ref[pl.ds(t * B, B), :] + _dot_t(h, whh_ref[...])
        i = jax.nn.sigmoid(g[:, :H])
        f = jax.nn.sigmoid(g[:, H:2 * H])
        gg = jnp.tanh(g[:, 2 * H:3 * H])
        o = jax.nn.sigmoid(g[:, 3 * H:])
        c = f * c + i * gg
        h = o * jnp.tanh(c)
        msk = (lens_ref[:, :1] - 1) == t
        hlast = jnp.where(msk, h, hlast)
        return (h, c, hlast)

    z = jnp.zeros((B, H), jnp.float32)
    _, _, hlast = lax.fori_loop(0, T, step, (z, z, z))
    pq_ref[...] = _dot_t(hlast, wq_vn[...]) + bq_vn[...]
    rq_ref[...] = _dot_t(hlast, wq_vr[...]) + bq_vr[...]
    sq_ref[...] = _dot_t(hlast, wq_sn[...]) + bq_sn[...]
    eq_ref[...] = _dot_t(hlast, wq_sr[...]) + bq_sr[...]
    fq_ref[...] = _dot_t(hlast, wq_fn[...]) + bq_fn[...]


def _lstm_call(qe, wihT, whh, bias, lens, wqs):
    return pl.pallas_call(
        _lstm_body,
        out_shape=[
            jax.ShapeDtypeStruct((B, PN), jnp.float32),
            jax.ShapeDtypeStruct((B, PR), jnp.float32),
            jax.ShapeDtypeStruct((B, PS), jnp.float32),
            jax.ShapeDtypeStruct((B, PR), jnp.float32),
            jax.ShapeDtypeStruct((B, PF), jnp.float32),
        ],
        scratch_shapes=[pltpu.VMEM((QTOT, 4 * H), jnp.float32)],
    )(qe, wihT, whh, bias, lens, *wqs)


def _mm_t_body(x_ref, w_ref, b_ref, o_ref):
    o_ref[...] = lax.dot_general(
        _bf(x_ref[...]), _bf(w_ref[...]), (((1,), (1,)), ((), ())),
        preferred_element_type=jnp.float32) + b_ref[...]


def _mm_n_body(x_ref, w_ref, b_ref, o_ref):
    o_ref[...] = lax.dot_general(
        _bf(x_ref[...]), _bf(w_ref[...]), (((1,), (0,)), ((), ())),
        preferred_element_type=jnp.float32) + b_ref[...]


def _proj(x, w, b, bm, transposed):
    m, k = x.shape
    n = w.shape[0] if transposed else w.shape[1]
    return pl.pallas_call(
        _mm_t_body if transposed else _mm_n_body,
        grid=(m // bm,),
        in_specs=[
            pl.BlockSpec((bm, k), lambda i: (i, 0)),
            pl.BlockSpec(w.shape, lambda i: (0, 0)),
            pl.BlockSpec((1, n), lambda i: (0, 0)),
        ],
        out_specs=pl.BlockSpec((bm, n), lambda i: (i, 0)),
        out_shape=jax.ShapeDtypeStruct((m, n), jnp.float32),
    )(x, w, b)


def _att_body(pq_ref, rq_ref, sq_ref, eq_ref, fq_ref,
              pi_ref, rel_ref, ss_ref, ee_ref, fn_ref,
              wr7, br_vr, bvr,
              w_vn, w_vr, w_sn, w_sr, w_fn,
              o_vn, o_vr, o_sn, o_sr, o_fn):
    def head(pq, proj, wv):
        s = jnp.tanh(pq[:, None, :] + proj)
        return jnp.sum(s * wv[None], axis=-1)

    def head_t(pq, proj_t, wv):
        s = jnp.tanh(pq[None, :, :] + proj_t)
        return jnp.sum(s * wv[None], axis=-1)

    def smax(x):
        m = jnp.max(x, axis=1, keepdims=True)
        e = jnp.exp(x - m)
        return e / jnp.sum(e, axis=1, keepdims=True)

    def smax_t(x):
        m = jnp.max(x, axis=0, keepdims=True)
        e = jnp.exp(x - m)
        return lax.transpose(e / jnp.sum(e, axis=0, keepdims=True), (1, 0))

    o_vn[...] = smax_t(head_t(pq_ref[...], pi_ref[...], w_vn[...]))
    o_sn[...] = smax(head(sq_ref[...], ss_ref[...], w_sn[...]))
    o_sr[...] = smax_t(head_t(eq_ref[...], ee_ref[...], w_sr[...]))
    o_fn[...] = smax_t(head_t(fq_ref[...], fn_ref[...], w_fn[...]))
    for bi in range(BB):
        rr = jnp.dot(rel_ref[bi], wr7[...],
                     preferred_element_type=jnp.float32) + br_vr[...]
        s = jnp.tanh(rq_ref[pl.ds(bi, 1), :] + rr)
        v = jnp.sum(s * w_vr[...], axis=-1, keepdims=True) + bvr[...]
        o_vr[pl.ds(bi, 1), :] = lax.transpose(v, (1, 0))


def _att_call(pqs, pi, rel, ss, ee, fn, ws):
    def bspec(shape):
        return pl.BlockSpec((BB,) + shape[1:],
                            lambda i: (i,) + (0,) * (len(shape) - 1))

    def tspec(shape):
        return pl.BlockSpec((shape[0], BB, shape[2]), lambda i: (0, i, 0))

    def cspec(shape):
        return pl.BlockSpec(shape, lambda i: (0,) * len(shape))

    ins = list(pqs) + [pi, rel, ss, ee, fn] + ws
    in_specs = [bspec(x.shape) for x in pqs]
    in_specs += [tspec(pi.shape), bspec(rel.shape), bspec(ss.shape),
                 tspec(ee.shape), tspec(fn.shape)]
    in_specs += [cspec(w.shape) for w in ws]
    out_shapes = [
        jax.ShapeDtypeStruct((B, R), jnp.float32),
        jax.ShapeDtypeStruct((B, R * R), jnp.float32),
        jax.ShapeDtypeStruct((B, SN), jnp.float32),
        jax.ShapeDtypeStruct((B, SE), jnp.float32),
        jax.ShapeDtypeStruct((B, FN), jnp.float32),
    ]
    out_specs = [bspec(s.shape) for s in out_shapes]
    return pl.pallas_call(
        _att_body,
        grid=(B // BB,),
        in_specs=in_specs,
        out_specs=out_specs,
        out_shape=out_shapes,
    )(*ins)


def kernel(questions, question_length, images, img_relations,
           sem_node_features, sem_edge_features, fact_node_features,
           glove, params):
    p = params

    idx = questions.astype(jnp.int32).T.reshape(-1)
    idx = jnp.concatenate([idx, jnp.zeros((QPAD - QTOT,), jnp.int32)])
    qe = _sc_gather_rows(_pad_glove_t(glove.T), idx)

    bias = (p["bih"] + p["bhh"]).reshape(1, 4 * H)
    lens = jnp.broadcast_to(
        question_length.astype(jnp.int32)[:, None], (B, 128))
    wqs = [
        p["vn_Wq"], p["vn_bq"].reshape(1, PN),
        p["vr_Wq"], p["vr_bq"].reshape(1, PR),
        p["sn_Wq"], p["sn_bq"].reshape(1, PS),
        p["sr_Wq"], p["sr_bq"].reshape(1, PR),
        p["fn_Wq"], p["fn_bq"].reshape(1, PF),
    ]
    pqs = list(_lstm_call(qe, p["Wih"].T, p["Whh"], bias, lens, wqs))

    pi = _proj(jnp.swapaxes(images, 0, 1).reshape(R * B, IMG), p["vn_Wi"],
               p["vn_bi"].reshape(1, PN), 128, True).reshape(R, B, PN)
    fn = _proj(jnp.swapaxes(fact_node_features, 0, 1).reshape(FN * B, FD),
               p["fn_Wn"], p["fn_bn"].reshape(1, PF), 128,
               True).reshape(FN, B, PF)
    ss = _proj(sem_node_features.reshape(B * SN, SD), p["sn_Ws"].T,
               p["sn_bs"].reshape(1, PS), 128, False).reshape(B, SN, PS)
    ee = _proj(jnp.swapaxes(sem_edge_features, 0, 1).reshape(SE * B, SRD),
               p["sr_Wr"].T, p["sr_br"].reshape(1, PR), 128,
               False).reshape(SE, B, PR)

    rel = img_relations.reshape(B, R * R, REL)

    ws = [
        p["vr_Wr"].T, p["vr_br"].reshape(1, PR), p["vr_b"].reshape(1, 1),
        p["vn_w"].reshape(1, PN), p["vr_w"].reshape(1, PR),
        p["sn_w"].reshape(1, PS), p["sr_w"].reshape(1, PR),
        p["fn_w"].reshape(1, PF),
    ]
    o_vn, o_vr, o_sn, o_sr, o_fn = _att_call(pqs, pi, rel, ss, ee, fn, ws)
    return o_vn, o_vr.reshape(B, R, R), o_sn, o_sr, o_fn


# speedup vs baseline: 2.1497x; 1.0252x over previous
import functools

import jax
import jax.numpy as jnp
from jax import lax
from jax.experimental import pallas as pl
from jax.experimental.pallas import tpu as pltpu
from jax.experimental.pallas import tpu_sc as plsc

B = 32; T = 20; G = 300; H = 512
R = 36; IMG = 2048; REL = 7
SN = 40; SD = 300; SE = 60; SRD = 300
FN = 100; FD = 1024
PN = 1024; PR = 512; PS = 512; PF = 512

NC, NS = 2, 16
NW = NC * NS
QTOT = B * T
PER_W = 24
QPAD = NW * PER_W
GP = 384
V = 20000
BB = 8


def _dot_t(x, w):
    return lax.dot_general(x, w, (((1,), (1,)), ((), ())),
                           preferred_element_type=jnp.float32)


def _bf(x):
    return x.astype(jnp.bfloat16)


def _pad_body(xt_ref, o_ref):
    blk = jnp.swapaxes(xt_ref[...], 0, 1)
    o_ref[...] = jnp.concatenate(
        [blk, jnp.zeros((blk.shape[0], GP - G), jnp.float32)], axis=1)


def _pad_glove_t(glove_t):
    bm = 512
    grid = (V + bm - 1) // bm
    return pl.pallas_call(
        _pad_body,
        grid=(grid,),
        in_specs=[pl.BlockSpec((G, bm), lambda i: (0, i))],
        out_specs=pl.BlockSpec((bm, GP), lambda i: (i, 0)),
        out_shape=jax.ShapeDtypeStruct((V, GP), jnp.float32),
    )(glove_t)


def _sc_gather_rows(glove_pad, idx):
    mesh = plsc.VectorSubcoreMesh(core_axis_name="c", subcore_axis_name="s")

    @functools.partial(
        pl.kernel, mesh=mesh,
        out_type=jax.ShapeDtypeStruct((QPAD, GP), jnp.float32),
        scratch_types=[
            pltpu.VMEM((PER_W,), jnp.int32),
            pltpu.VMEM((PER_W, GP), jnp.float32),
            pltpu.SemaphoreType.DMA,
        ],
    )
    def gather_k(glove_hbm, idx_hbm, out_hbm, idx_v, rows_v, sem):
        wid = lax.axis_index("s") * NC + lax.axis_index("c")
        base = wid * PER_W
        pltpu.sync_copy(idx_hbm.at[pl.ds(base, PER_W)], idx_v)
        pltpu.async_copy(glove_hbm.at[idx_v], rows_v, sem).wait()
        pltpu.sync_copy(rows_v, out_hbm.at[pl.ds(base, PER_W)])

    return gather_k(glove_pad, idx)


def _lstm_body(qe_ref, wihT_ref, whh_ref, bias_ref, lens_ref,
               wq_vn, bq_vn, wq_vr, bq_vr, wq_sn, bq_sn,
               wq_sr, bq_sr, wq_fn, bq_fn,
               pq_ref, rq_ref, sq_ref, eq_ref, fq_ref, xg_ref):
    xg_ref[...] = (
        jnp.dot(qe_ref[pl.ds(0, QTOT), pl.ds(0, G)], wihT_ref[...],
                preferred_element_type=jnp.float32)
        + bias_ref[...]
    )

    def step(t, carry):
        h, c, hlast = carry
        g = xg_ref[pl.ds(t * B, B), :] + _dot_t(h, whh_ref[...])
        i = jax.nn.sigmoid(g[:, :H])
        f = jax.nn.sigmoid(g[:, H:2 * H])
        gg = jnp.tanh(g[:, 2 * H:3 * H])
        o = jax.nn.sigmoid(g[:, 3 * H:])
        c = f * c + i * gg
        h = o * jnp.tanh(c)
        msk = (lens_ref[:, :1] - 1) == t
        hlast = jnp.where(msk, h, hlast)
        return (h, c, hlast)

    z = jnp.zeros((B, H), jnp.float32)
    _, _, hlast = lax.fori_loop(0, T, step, (z, z, z))
    pq_ref[...] = _dot_t(hlast, wq_vn[...]) + bq_vn[...]
    rq_ref[...] = _dot_t(hlast, wq_vr[...]) + bq_vr[...]
    sq_ref[...] = _dot_t(hlast, wq_sn[...]) + bq_sn[...]
    eq_ref[...] = _dot_t(hlast, wq_sr[...]) + bq_sr[...]
    fq_ref[...] = _dot_t(hlast, wq_fn[...]) + bq_fn[...]


def _lstm_call(qe, wihT, whh, bias, lens, wqs):
    return pl.pallas_call(
        _lstm_body,
        out_shape=[
            jax.ShapeDtypeStruct((B, PN), jnp.float32),
            jax.ShapeDtypeStruct((B, PR), jnp.float32),
            jax.ShapeDtypeStruct((B, PS), jnp.float32),
            jax.ShapeDtypeStruct((B, PR), jnp.float32),
            jax.ShapeDtypeStruct((B, PF), jnp.float32),
        ],
        scratch_shapes=[pltpu.VMEM((QTOT, 4 * H), jnp.float32)],
    )(qe, wihT, whh, bias, lens, *wqs)


def _mm_t_body(x_ref, w_ref, b_ref, o_ref):
    o_ref[...] = lax.dot_general(
        _bf(x_ref[...]), _bf(w_ref[...]), (((1,), (1,)), ((), ())),
        preferred_element_type=jnp.float32) + b_ref[...]


def _mm_n_body(x_ref, w_ref, b_ref, o_ref):
    o_ref[...] = lax.dot_general(
        _bf(x_ref[...]), _bf(w_ref[...]), (((1,), (0,)), ((), ())),
        preferred_element_type=jnp.float32) + b_ref[...]


def _proj(x, w, b, bm, transposed):
    m, k = x.shape
    n = w.shape[0] if transposed else w.shape[1]
    return pl.pallas_call(
        _mm_t_body if transposed else _mm_n_body,
        grid=(m // bm,),
        in_specs=[
            pl.BlockSpec((bm, k), lambda i: (i, 0)),
            pl.BlockSpec(w.shape, lambda i: (0, 0)),
            pl.BlockSpec((1, n), lambda i: (0, 0)),
        ],
        out_specs=pl.BlockSpec((bm, n), lambda i: (i, 0)),
        out_shape=jax.ShapeDtypeStruct((m, n), jnp.float32),
    )(x, w, b)


def _att_body(pq_ref, rq_ref, sq_ref, eq_ref, fq_ref,
              pi_ref, rel_ref, ss_ref, ee_ref, fn_ref,
              wr7, br_vr, bvr,
              w_vn, w_vr, w_sn, w_sr, w_fn,
              o_vn, o_vr, o_sn, o_sr, o_fn):
    def head(pq, proj, wv):
        s = jnp.tanh(pq[:, None, :] + proj)
        return jnp.sum(s * wv[None], axis=-1)

    def head_t(pq, proj_t, wv):
        s = jnp.tanh(pq[None, :, :] + proj_t)
        return jnp.sum(s * wv[None], axis=-1)

    def smax(x):
        m = jnp.max(x, axis=1, keepdims=True)
        e = jnp.exp(x - m)
        return e / jnp.sum(e, axis=1, keepdims=True)

    def smax_t(x):
        m = jnp.max(x, axis=0, keepdims=True)
        e = jnp.exp(x - m)
        return lax.transpose(e / jnp.sum(e, axis=0, keepdims=True), (1, 0))

    o_vn[...] = smax_t(head_t(pq_ref[...], pi_ref[...], w_vn[...]))
    o_sn[...] = smax(head(sq_ref[...], ss_ref[...], w_sn[...]))
    o_sr[...] = smax_t(head_t(eq_ref[...], ee_ref[...], w_sr[...]))
    o_fn[...] = smax_t(head_t(fq_ref[...], fn_ref[...], w_fn[...]))
    rel4 = rel_ref[...]
    for bi in range(BB):
        xi = jnp.swapaxes(rel4[:, :, bi, :], 0, 1)
        xt = lax.transpose(xi.reshape(REL, R * R), (1, 0))
        rr = jnp.dot(xt, wr7[...],
                     preferred_element_type=jnp.float32) + br_vr[...]
        s = jnp.tanh(rq_ref[pl.ds(bi, 1), :] + rr)
        v = jnp.sum(s * w_vr[...], axis=-1, keepdims=True) + bvr[...]
        o_vr[pl.ds(bi, 1), :] = lax.transpose(v, (1, 0))


def _att_call(pqs, pi, rel, ss, ee, fn, ws):
    def bspec(shape):
        return pl.BlockSpec((BB,) + shape[1:],
                            lambda i: (i,) + (0,) * (len(shape) - 1))

    def tspec(shape):
        return pl.BlockSpec((shape[0], BB, shape[2]), lambda i: (0, i, 0))

    def cspec(shape):
        return pl.BlockSpec(shape, lambda i: (0,) * len(shape))

    ins = list(pqs) + [pi, rel, ss, ee, fn] + ws
    in_specs = [bspec(x.shape) for x in pqs]
    in_specs += [tspec(pi.shape),
                 pl.BlockSpec((R, REL, BB, R), lambda i: (0, 0, i, 0)),
                 bspec(ss.shape), tspec(ee.shape), tspec(fn.shape)]
    in_specs += [cspec(w.shape) for w in ws]
    out_shapes = [
        jax.ShapeDtypeStruct((B, R), jnp.float32),
        jax.ShapeDtypeStruct((B, R * R), jnp.float32),
        jax.ShapeDtypeStruct((B, SN), jnp.float32),
        jax.ShapeDtypeStruct((B, SE), jnp.float32),
        jax.ShapeDtypeStruct((B, FN), jnp.float32),
    ]
    out_specs = [bspec(s.shape) for s in out_shapes]
    return pl.pallas_call(
        _att_body,
        grid=(B // BB,),
        in_specs=in_specs,
        out_specs=out_specs,
        out_shape=out_shapes,
    )(*ins)


def kernel(questions, question_length, images, img_relations,
           sem_node_features, sem_edge_features, fact_node_features,
           glove, params):
    p = params

    idx = questions.astype(jnp.int32).T.reshape(-1)
    idx = jnp.concatenate([idx, jnp.zeros((QPAD - QTOT,), jnp.int32)])
    qe = _sc_gather_rows(_pad_glove_t(glove.T), idx)

    bias = (p["bih"] + p["bhh"]).reshape(1, 4 * H)
    lens = jnp.broadcast_to(
        question_length.astype(jnp.int32)[:, None], (B, 128))
    wqs = [
        p["vn_Wq"], p["vn_bq"].reshape(1, PN),
        p["vr_Wq"], p["vr_bq"].reshape(1, PR),
        p["sn_Wq"], p["sn_bq"].reshape(1, PS),
        p["sr_Wq"], p["sr_bq"].reshape(1, PR),
        p["fn_Wq"], p["fn_bq"].reshape(1, PF),
    ]
    pqs = list(_lstm_call(qe, p["Wih"].T, p["Whh"], bias, lens, wqs))

    pi = _proj(jnp.swapaxes(images, 0, 1).reshape(R * B, IMG), p["vn_Wi"],
               p["vn_bi"].reshape(1, PN), 128, True).reshape(R, B, PN)
    fn = _proj(jnp.swapaxes(fact_node_features, 0, 1).reshape(FN * B, FD),
               p["fn_Wn"], p["fn_bn"].reshape(1, PF), 128,
               True).reshape(FN, B, PF)
    ss = _proj(sem_node_features.reshape(B * SN, SD), p["sn_Ws"].T,
               p["sn_bs"].reshape(1, PS), 128, False).reshape(B, SN, PS)
    ee = _proj(jnp.swapaxes(sem_edge_features, 0, 1).reshape(SE * B, SRD),
               p["sr_Wr"].T, p["sr_br"].reshape(1, PR), 128,
               False).reshape(SE, B, PR)

    rel = jnp.transpose(img_relations, (1, 3, 0, 2))

    ws = [
        p["vr_Wr"].T, p["vr_br"].reshape(1, PR), p["vr_b"].reshape(1, 1),
        p["vn_w"].reshape(1, PN), p["vr_w"].reshape(1, PR),
        p["sn_w"].reshape(1, PS), p["sr_w"].reshape(1, PR),
        p["fn_w"].reshape(1, PF),
    ]
    o_vn, o_vr, o_sn, o_sr, o_fn = _att_call(pqs, pi, rel, ss, ee, fn, ws)
    return o_vn, o_vr.reshape(B, R, R), o_sn, o_sr, o_fn
